# baseline (speedup 1.0000x reference)
import functools

import jax
import jax.numpy as jnp
from jax import lax
from jax.experimental import pallas as pl
from jax.experimental.pallas import tpu as pltpu
from jax.experimental.pallas import tpu_sc as plsc

N_NODES = 10000
N_EDGES = 320000
D = 128
LANES = 16

N_TILES = 16
CHUNK = 128
GROUP = 8
G0 = 20
G1 = 0
N_CHUNKS0 = G0 * GROUP
N_CHUNKS1 = G1 * GROUP
E_SPLIT = N_TILES * N_CHUNKS0 * CHUNK
E_PAD = N_TILES * (N_CHUNKS0 + N_CHUNKS1) * CHUNK
N_ACT = 1 if G1 == 0 else 2
N_PAD = 10240
ROWS_PER_TILE = N_PAD // N_TILES


def _sc_body(with_cnt, x_hbm, *refs):
  idx_hbm = refs[:2 * N_ACT]
  refs = refs[2 * N_ACT:]
  if with_cnt:
    out_hbm, cnt_hbm, agg_sh, cnt_sh, cbuf, rbuf, gbuf, gbuf2, gsem, obuf = refs
  else:
    out_hbm, agg_sh, cbuf, rbuf, gbuf, gbuf2, gsem = refs

  c = lax.axis_index("c")
  s = lax.axis_index("s")
  slab = s * ROWS_PER_TILE
  gbufs = (gbuf, gbuf2)

  def core_work(core_id, rows_hbm, cols_hbm, n_groups):
    def zrow(r, _):
      for k in range(D // LANES):
        gbuf[r, pl.ds(k * LANES, LANES)] = jnp.zeros((LANES,), jnp.float32)
      return 0
    lax.fori_loop(0, CHUNK, zrow, 0)
    for k in range(ROWS_PER_TILE // CHUNK):
      pltpu.sync_copy(gbuf, agg_sh.at[pl.ds(slab + k * CHUNK, CHUNK)])

    if with_cnt:
      def zrow2(r, _):
        obuf[r] = jnp.zeros((LANES,), jnp.float32)
        return 0
      lax.fori_loop(0, CHUNK, zrow2, 0)
      for k in range(ROWS_PER_TILE // CHUNK):
        pltpu.sync_copy(obuf, cnt_sh.at[pl.ds(slab + k * CHUNK, CHUNK)])
      def orow(r, _):
        obuf[r] = jnp.ones((LANES,), jnp.float32)
        return 0
      lax.fori_loop(0, CHUNK, orow, 0)

    plsc.subcore_barrier()

    def group(g, _):
      pltpu.sync_copy(cols_hbm.at[s, pl.ds(g * GROUP, GROUP)], cbuf)
      pltpu.sync_copy(rows_hbm.at[s, pl.ds(g * GROUP, GROUP)], rbuf)

      h = pltpu.async_copy(x_hbm.at[cbuf.at[0]], gbufs[0], gsem)
      for j in range(GROUP):
        h.wait()
        if j + 1 < GROUP:
          h = pltpu.async_copy(x_hbm.at[cbuf.at[j + 1]], gbufs[(j + 1) % 2],
                               gsem)
        pltpu.sync_copy(gbufs[j % 2], agg_sh.at[rbuf.at[j]], add=True)
        if with_cnt:
          pltpu.sync_copy(obuf, cnt_sh.at[rbuf.at[j]], add=True)
      return 0
    lax.fori_loop(0, n_groups, group, 0)

    plsc.subcore_barrier()
    pltpu.sync_copy(agg_sh.at[pl.ds(slab, ROWS_PER_TILE)],
                    out_hbm.at[core_id, pl.ds(slab, ROWS_PER_TILE)])
    if with_cnt:
      pltpu.sync_copy(cnt_sh.at[pl.ds(slab, ROWS_PER_TILE)],
                      cnt_hbm.at[core_id, pl.ds(slab, ROWS_PER_TILE)])

  @pl.when(c == 0)
  def _():
    core_work(0, idx_hbm[0], idx_hbm[1], G0)

  if N_ACT == 2:
    @pl.when(c == 1)
    def _():
      core_work(1, idx_hbm[2], idx_hbm[3], G1)


def _make_sc_pass(with_cnt):
  out_types = [jax.ShapeDtypeStruct((N_ACT, N_PAD, D), jnp.float32)]
  scratch = [pltpu.VMEM_SHARED((N_PAD, D), jnp.float32)]
  if with_cnt:
    out_types.append(jax.ShapeDtypeStruct((N_ACT, N_PAD, LANES), jnp.float32))
    scratch.append(pltpu.VMEM_SHARED((N_PAD, LANES), jnp.float32))
  scratch += [
      pltpu.VMEM((GROUP, CHUNK), jnp.int32),
      pltpu.VMEM((GROUP, CHUNK), jnp.int32),
      pltpu.VMEM((CHUNK, D), jnp.float32),
      pltpu.VMEM((CHUNK, D), jnp.float32),
      pltpu.SemaphoreType.DMA,
  ]
  if with_cnt:
    scratch.append(pltpu.VMEM((CHUNK, LANES), jnp.float32))
  mesh = plsc.VectorSubcoreMesh(core_axis_name="c", subcore_axis_name="s")
  return pl.kernel(
      functools.partial(_sc_body, with_cnt),
      out_type=tuple(out_types) if with_cnt else out_types[0],
      mesh=mesh,
      scratch_types=scratch,
      compiler_params=pltpu.CompilerParams(use_tc_tiling_on_sc=False),
  )


BLK = 2000


def _conv_body(x_ref, p_ref, c_ref, wt_ref, wb_ref, b_ref, o_ref):
  cnt = jnp.sum(c_ref[...], axis=0)[:, 0:1]
  agg = jnp.sum(p_ref[...], axis=0)
  num = cnt * (jnp.dot(x_ref[...], wt_ref[...],
                       preferred_element_type=jnp.float32) + b_ref[...])
  num = num + jnp.dot(agg, wb_ref[...], preferred_element_type=jnp.float32)
  o_ref[...] = jnp.maximum(num / jnp.maximum(cnt, 1.0), 0.0)


def _final_body(x_ref, x1_ref, p_ref, c_ref, wt_ref, wb_ref, b_ref,
                wa_ref, wbb_ref, wc_ref, bl1_ref, wl2_ref, bl2_ref, o_ref):
  cnt = jnp.sum(c_ref[...], axis=0)[:, 0:1]
  agg = jnp.sum(p_ref[...], axis=0)
  num = cnt * (jnp.dot(x1_ref[...], wt_ref[...],
                       preferred_element_type=jnp.float32) + b_ref[...])
  num = num + jnp.dot(agg, wb_ref[...], preferred_element_type=jnp.float32)
  x2 = jnp.maximum(num / jnp.maximum(cnt, 1.0), 0.0)
  h = (jnp.dot(x_ref[...], wa_ref[...], preferred_element_type=jnp.float32)
       + jnp.dot(x1_ref[...], wbb_ref[...], preferred_element_type=jnp.float32)
       + jnp.dot(x2, wc_ref[...], preferred_element_type=jnp.float32)
       + bl1_ref[...])
  h = jnp.maximum(h, 0.0)
  logits = jnp.dot(h, wl2_ref[...], preferred_element_type=jnp.float32) + bl2_ref[...]
  o_ref[...] = jax.nn.sigmoid(logits)


def _row_spec(shape):
  return pl.BlockSpec(shape, lambda i: (i, 0))


def _part_spec(shape):
  return pl.BlockSpec(shape, lambda i: (0, i, 0))


def _full_spec(shape):
  return pl.BlockSpec(shape, lambda i: (0,) * len(shape))


def kernel(x, edge_index, W1, b1, W2, b2, Wl1, bl1, Wl2, bl2):
  rows = edge_index[0]
  cols = edge_index[1]
  pad = E_PAD - N_EDGES
  pad_rows = N_NODES + (jnp.arange(pad, dtype=jnp.int32) % (N_PAD - N_NODES))
  rows_p = jnp.concatenate([rows, pad_rows])
  cols_p = jnp.concatenate([cols, jnp.zeros((pad,), jnp.int32)])
  idx = [
      rows_p[:E_SPLIT].reshape(N_TILES, N_CHUNKS0, CHUNK),
      cols_p[:E_SPLIT].reshape(N_TILES, N_CHUNKS0, CHUNK),
  ]
  if N_ACT == 2:
    idx += [
        rows_p[E_SPLIT:].reshape(N_TILES, N_CHUNKS1, CHUNK),
        cols_p[E_SPLIT:].reshape(N_TILES, N_CHUNKS1, CHUNK),
    ]

  sc_pass1 = _make_sc_pass(True)
  sc_pass2 = _make_sc_pass(False)

  p1, c1 = sc_pass1(x, *idx)
  grid = (N_NODES // BLK,)

  x1 = pl.pallas_call(
      _conv_body,
      grid=grid,
      in_specs=[
          _row_spec((BLK, D)),
          _part_spec((N_ACT, BLK, D)),
          _part_spec((N_ACT, BLK, LANES)),
          _full_spec((D, D)),
          _full_spec((D, D)),
          _full_spec((1, D)),
      ],
      out_specs=_row_spec((BLK, D)),
      out_shape=jax.ShapeDtypeStruct((N_NODES, D), jnp.float32),
  )(x, p1, c1, W1[:D], W1[D:], b1.reshape(1, D))

  p2 = sc_pass2(x1, *idx)

  out = pl.pallas_call(
      _final_body,
      grid=grid,
      in_specs=[
          _row_spec((BLK, D)),
          _row_spec((BLK, D)),
          _part_spec((N_ACT, BLK, D)),
          _part_spec((N_ACT, BLK, LANES)),
          _full_spec((D, D)),
          _full_spec((D, D)),
          _full_spec((1, D)),
          _full_spec((D, D)),
          _full_spec((D, D)),
          _full_spec((D, D)),
          _full_spec((1, D)),
          _full_spec((D, D)),
          _full_spec((1, D)),
      ],
      out_specs=_row_spec((BLK, D)),
      out_shape=jax.ShapeDtypeStruct((N_NODES, D), jnp.float32),
  )(x, x1, p2, c1, W2[:D], W2[D:], b2.reshape(1, D),
    Wl1[:D], Wl1[D:2 * D], Wl1[2 * D:], bl1.reshape(1, D),
    Wl2, bl2.reshape(1, D))

  return out


# speedup vs baseline: 1.2067x; 1.2067x over previous
import functools

import jax
import jax.numpy as jnp
from jax import lax
from jax.experimental import pallas as pl
from jax.experimental.pallas import tpu as pltpu
from jax.experimental.pallas import tpu_sc as plsc

N_NODES = 10000
N_EDGES = 320000
D = 128
LANES = 16

N_TILES = 16
CHUNK = 128
GROUP = 8
G0 = 16
G1 = 4
N_CHUNKS0 = G0 * GROUP
N_CHUNKS1 = G1 * GROUP
E_SPLIT = N_TILES * N_CHUNKS0 * CHUNK
E_PAD = N_TILES * (N_CHUNKS0 + N_CHUNKS1) * CHUNK
N_ACT = 1 if G1 == 0 else 2
N_PAD = 10240
ROWS_PER_TILE = N_PAD // N_TILES


def _sc_body(with_cnt, x_hbm, *refs):
  idx_hbm = refs[:2 * N_ACT]
  refs = refs[2 * N_ACT:]
  if with_cnt:
    out_hbm, cnt_hbm, agg_sh, cnt_sh, cbuf, rbuf, gbuf, gbuf2, gsem, obuf = refs
  else:
    out_hbm, agg_sh, cbuf, rbuf, gbuf, gbuf2, gsem = refs

  c = lax.axis_index("c")
  s = lax.axis_index("s")
  slab = s * ROWS_PER_TILE
  gbufs = (gbuf, gbuf2)

  def core_work(core_id, rows_hbm, cols_hbm, n_groups):
    def zrow(r, _):
      for k in range(D // LANES):
        gbuf[r, pl.ds(k * LANES, LANES)] = jnp.zeros((LANES,), jnp.float32)
      return 0
    lax.fori_loop(0, CHUNK, zrow, 0)
    for k in range(ROWS_PER_TILE // CHUNK):
      pltpu.sync_copy(gbuf, agg_sh.at[pl.ds(slab + k * CHUNK, CHUNK)])

    if with_cnt:
      def zrow2(r, _):
        obuf[r] = jnp.zeros((LANES,), jnp.float32)
        return 0
      lax.fori_loop(0, CHUNK, zrow2, 0)
      for k in range(ROWS_PER_TILE // CHUNK):
        pltpu.sync_copy(obuf, cnt_sh.at[pl.ds(slab + k * CHUNK, CHUNK)])
      def orow(r, _):
        obuf[r] = jnp.ones((LANES,), jnp.float32)
        return 0
      lax.fori_loop(0, CHUNK, orow, 0)

    plsc.subcore_barrier()

    def group(g, _):
      pltpu.sync_copy(cols_hbm.at[s, pl.ds(g * GROUP, GROUP)], cbuf)
      pltpu.sync_copy(rows_hbm.at[s, pl.ds(g * GROUP, GROUP)], rbuf)

      h = pltpu.async_copy(x_hbm.at[cbuf.at[0]], gbufs[0], gsem)
      for j in range(GROUP):
        h.wait()
        if j + 1 < GROUP:
          h = pltpu.async_copy(x_hbm.at[cbuf.at[j + 1]], gbufs[(j + 1) % 2],
                               gsem)
        pltpu.sync_copy(gbufs[j % 2], agg_sh.at[rbuf.at[j]], add=True)
        if with_cnt:
          pltpu.sync_copy(obuf, cnt_sh.at[rbuf.at[j]], add=True)
      return 0
    lax.fori_loop(0, n_groups, group, 0)

    plsc.subcore_barrier()
    pltpu.sync_copy(agg_sh.at[pl.ds(slab, ROWS_PER_TILE)],
                    out_hbm.at[core_id, pl.ds(slab, ROWS_PER_TILE)])
    if with_cnt:
      pltpu.sync_copy(cnt_sh.at[pl.ds(slab, ROWS_PER_TILE)],
                      cnt_hbm.at[core_id, pl.ds(slab, ROWS_PER_TILE)])

  @pl.when(c == 0)
  def _():
    core_work(0, idx_hbm[0], idx_hbm[1], G0)

  if N_ACT == 2:
    @pl.when(c == 1)
    def _():
      core_work(1, idx_hbm[2], idx_hbm[3], G1)


def _make_sc_pass(with_cnt):
  out_types = [jax.ShapeDtypeStruct((N_ACT, N_PAD, D), jnp.float32)]
  scratch = [pltpu.VMEM_SHARED((N_PAD, D), jnp.float32)]
  if with_cnt:
    out_types.append(jax.ShapeDtypeStruct((N_ACT, N_PAD, LANES), jnp.float32))
    scratch.append(pltpu.VMEM_SHARED((N_PAD, LANES), jnp.float32))
  scratch += [
      pltpu.VMEM((GROUP, CHUNK), jnp.int32),
      pltpu.VMEM((GROUP, CHUNK), jnp.int32),
      pltpu.VMEM((CHUNK, D), jnp.float32),
      pltpu.VMEM((CHUNK, D), jnp.float32),
      pltpu.SemaphoreType.DMA,
  ]
  if with_cnt:
    scratch.append(pltpu.VMEM((CHUNK, LANES), jnp.float32))
  mesh = plsc.VectorSubcoreMesh(core_axis_name="c", subcore_axis_name="s")
  return pl.kernel(
      functools.partial(_sc_body, with_cnt),
      out_type=tuple(out_types) if with_cnt else out_types[0],
      mesh=mesh,
      scratch_types=scratch,
      compiler_params=pltpu.CompilerParams(use_tc_tiling_on_sc=False),
  )


BLK = 2000


def _conv_body(x_ref, p_ref, c_ref, wt_ref, wb_ref, b_ref, o_ref):
  cnt = jnp.sum(c_ref[...], axis=0)[:, 0:1]
  agg = jnp.sum(p_ref[...], axis=0)
  num = cnt * (jnp.dot(x_ref[...], wt_ref[...],
                       preferred_element_type=jnp.float32) + b_ref[...])
  num = num + jnp.dot(agg, wb_ref[...], preferred_element_type=jnp.float32)
  o_ref[...] = jnp.maximum(num / jnp.maximum(cnt, 1.0), 0.0)


def _final_body(x_ref, x1_ref, p_ref, c_ref, wt_ref, wb_ref, b_ref,
                wa_ref, wbb_ref, wc_ref, bl1_ref, wl2_ref, bl2_ref, o_ref):
  cnt = jnp.sum(c_ref[...], axis=0)[:, 0:1]
  agg = jnp.sum(p_ref[...], axis=0)
  num = cnt * (jnp.dot(x1_ref[...], wt_ref[...],
                       preferred_element_type=jnp.float32) + b_ref[...])
  num = num + jnp.dot(agg, wb_ref[...], preferred_element_type=jnp.float32)
  x2 = jnp.maximum(num / jnp.maximum(cnt, 1.0), 0.0)
  h = (jnp.dot(x_ref[...], wa_ref[...], preferred_element_type=jnp.float32)
       + jnp.dot(x1_ref[...], wbb_ref[...], preferred_element_type=jnp.float32)
       + jnp.dot(x2, wc_ref[...], preferred_element_type=jnp.float32)
       + bl1_ref[...])
  h = jnp.maximum(h, 0.0)
  logits = jnp.dot(h, wl2_ref[...], preferred_element_type=jnp.float32) + bl2_ref[...]
  o_ref[...] = jax.nn.sigmoid(logits)


def _row_spec(shape):
  return pl.BlockSpec(shape, lambda i: (i, 0))


def _part_spec(shape):
  return pl.BlockSpec(shape, lambda i: (0, i, 0))


def _full_spec(shape):
  return pl.BlockSpec(shape, lambda i: (0,) * len(shape))


def kernel(x, edge_index, W1, b1, W2, b2, Wl1, bl1, Wl2, bl2):
  rows = edge_index[0]
  cols = edge_index[1]
  pad = E_PAD - N_EDGES
  pad_rows = N_NODES + (jnp.arange(pad, dtype=jnp.int32) % (N_PAD - N_NODES))
  rows_p = jnp.concatenate([rows, pad_rows])
  cols_p = jnp.concatenate([cols, jnp.zeros((pad,), jnp.int32)])
  idx = [
      rows_p[:E_SPLIT].reshape(N_TILES, N_CHUNKS0, CHUNK),
      cols_p[:E_SPLIT].reshape(N_TILES, N_CHUNKS0, CHUNK),
  ]
  if N_ACT == 2:
    idx += [
        rows_p[E_SPLIT:].reshape(N_TILES, N_CHUNKS1, CHUNK),
        cols_p[E_SPLIT:].reshape(N_TILES, N_CHUNKS1, CHUNK),
    ]

  sc_pass1 = _make_sc_pass(True)
  sc_pass2 = _make_sc_pass(False)

  p1, c1 = sc_pass1(x, *idx)
  grid = (N_NODES // BLK,)

  x1 = pl.pallas_call(
      _conv_body,
      grid=grid,
      in_specs=[
          _row_spec((BLK, D)),
          _part_spec((N_ACT, BLK, D)),
          _part_spec((N_ACT, BLK, LANES)),
          _full_spec((D, D)),
          _full_spec((D, D)),
          _full_spec((1, D)),
      ],
      out_specs=_row_spec((BLK, D)),
      out_shape=jax.ShapeDtypeStruct((N_NODES, D), jnp.float32),
  )(x, p1, c1, W1[:D], W1[D:], b1.reshape(1, D))

  p2 = sc_pass2(x1, *idx)

  out = pl.pallas_call(
      _final_body,
      grid=grid,
      in_specs=[
          _row_spec((BLK, D)),
          _row_spec((BLK, D)),
          _part_spec((N_ACT, BLK, D)),
          _part_spec((N_ACT, BLK, LANES)),
          _full_spec((D, D)),
          _full_spec((D, D)),
          _full_spec((1, D)),
          _full_spec((D, D)),
          _full_spec((D, D)),
          _full_spec((D, D)),
          _full_spec((1, D)),
          _full_spec((D, D)),
          _full_spec((1, D)),
      ],
      out_specs=_row_spec((BLK, D)),
      out_shape=jax.ShapeDtypeStruct((N_NODES, D), jnp.float32),
  )(x, x1, p2, c1, W2[:D], W2[D:], b2.reshape(1, D),
    Wl1[:D], Wl1[D:2 * D], Wl1[2 * D:], bl1.reshape(1, D),
    Wl2, bl2.reshape(1, D))

  return out


# speedup vs baseline: 2.3722x; 1.9658x over previous
import functools

import jax
import jax.numpy as jnp
from jax import lax
from jax.experimental import pallas as pl
from jax.experimental.pallas import tpu as pltpu
from jax.experimental.pallas import tpu_sc as plsc

N_NODES = 10000
N_EDGES = 320000
D = 128
LANES = 16

N_TILES = 16
CHUNK = 128
GROUP = 8
G0 = 16
G1 = 4
N_CHUNKS0 = G0 * GROUP
N_CHUNKS1 = G1 * GROUP
E_SPLIT = N_TILES * N_CHUNKS0 * CHUNK
E_PAD = N_TILES * (N_CHUNKS0 + N_CHUNKS1) * CHUNK
N_ACT = 1 if G1 == 0 else 2
N_PAD = 10240
ROWS_PER_TILE = N_PAD // N_TILES


def _sc_body(with_cnt, x_hbm, *refs):
  idx_hbm = refs[:2 * N_ACT]
  refs = refs[2 * N_ACT:]
  if with_cnt:
    out_hbm, cnt_hbm, agg_sh, cnt_sh, cbuf, rbuf, gbuf, gbuf2, gsem, obuf = refs
  else:
    out_hbm, agg_sh, cbuf, rbuf, gbuf, gbuf2, gsem = refs

  c = lax.axis_index("c")
  s = lax.axis_index("s")
  slab = s * ROWS_PER_TILE
  gbufs = (gbuf, gbuf2)

  def core_work(core_id, rows_hbm, cols_hbm, n_groups):
    def zrow(r, _):
      for k in range(D // LANES):
        gbuf[r, pl.ds(k * LANES, LANES)] = jnp.zeros((LANES,), jnp.float32)
      return 0
    lax.fori_loop(0, CHUNK, zrow, 0)
    for k in range(ROWS_PER_TILE // CHUNK):
      pltpu.sync_copy(gbuf, agg_sh.at[pl.ds(slab + k * CHUNK, CHUNK)])

    if with_cnt:
      def zrow2(r, _):
        obuf[r] = jnp.zeros((LANES,), jnp.float32)
        return 0
      lax.fori_loop(0, CHUNK, zrow2, 0)
      for k in range(ROWS_PER_TILE // CHUNK):
        pltpu.sync_copy(obuf, cnt_sh.at[pl.ds(slab + k * CHUNK, CHUNK)])
      def orow(r, _):
        obuf[r] = jnp.ones((LANES,), jnp.float32)
        return 0
      lax.fori_loop(0, CHUNK, orow, 0)

    plsc.subcore_barrier()

    def group(g, _):
      pltpu.sync_copy(cols_hbm.at[s, pl.ds(g * GROUP, GROUP)], cbuf)
      pltpu.sync_copy(rows_hbm.at[s, pl.ds(g * GROUP, GROUP)], rbuf)

      h = pltpu.async_copy(x_hbm.at[cbuf.at[0]], gbufs[0], gsem)
      for j in range(GROUP):
        h.wait()
        if j + 1 < GROUP:
          h = pltpu.async_copy(x_hbm.at[cbuf.at[j + 1]], gbufs[(j + 1) % 2],
                               gsem)
        pltpu.sync_copy(gbufs[j % 2], agg_sh.at[rbuf.at[j]], add=True)
        if with_cnt:
          pltpu.sync_copy(obuf, cnt_sh.at[rbuf.at[j]], add=True)
      return 0
    lax.fori_loop(0, n_groups, group, 0)

    plsc.subcore_barrier()
    pltpu.sync_copy(agg_sh.at[pl.ds(slab, ROWS_PER_TILE)],
                    out_hbm.at[core_id, pl.ds(slab, ROWS_PER_TILE)])
    if with_cnt:
      pltpu.sync_copy(cnt_sh.at[pl.ds(slab, ROWS_PER_TILE)],
                      cnt_hbm.at[core_id, pl.ds(slab, ROWS_PER_TILE)])

  @pl.when(c == 0)
  def _():
    core_work(0, idx_hbm[0], idx_hbm[1], G0)

  if N_ACT == 2:
    @pl.when(c == 1)
    def _():
      core_work(1, idx_hbm[2], idx_hbm[3], G1)


def _make_sc_pass(with_cnt):
  out_types = [jax.ShapeDtypeStruct((N_ACT, N_PAD, D), jnp.float32)]
  scratch = [pltpu.VMEM_SHARED((N_PAD, D), jnp.float32)]
  if with_cnt:
    out_types.append(jax.ShapeDtypeStruct((N_ACT, N_PAD, LANES), jnp.float32))
    scratch.append(pltpu.VMEM_SHARED((N_PAD, LANES), jnp.float32))
  scratch += [
      pltpu.VMEM((GROUP, CHUNK), jnp.int32),
      pltpu.VMEM((GROUP, CHUNK), jnp.int32),
      pltpu.VMEM((CHUNK, D), jnp.float32),
      pltpu.VMEM((CHUNK, D), jnp.float32),
      pltpu.SemaphoreType.DMA,
  ]
  if with_cnt:
    scratch.append(pltpu.VMEM((CHUNK, LANES), jnp.float32))
  mesh = plsc.VectorSubcoreMesh(core_axis_name="c", subcore_axis_name="s")
  return pl.kernel(
      functools.partial(_sc_body, with_cnt),
      out_type=tuple(out_types) if with_cnt else out_types[0],
      mesh=mesh,
      scratch_types=scratch,
      compiler_params=pltpu.CompilerParams(use_tc_tiling_on_sc=False),
  )


BLK = 2000


def _conv_body(x_ref, p_ref, c_ref, wt_ref, wb_ref, b_ref, o_ref):
  cnt = jnp.sum(c_ref[...], axis=0)[:, 0:1]
  agg = jnp.sum(p_ref[...], axis=0)
  num = cnt * (jnp.dot(x_ref[...], wt_ref[...],
                       preferred_element_type=jnp.float32) + b_ref[...])
  num = num + jnp.dot(agg, wb_ref[...], preferred_element_type=jnp.float32)
  o_ref[...] = jnp.maximum(num / jnp.maximum(cnt, 1.0), 0.0)


def _final_body(x_ref, x1_ref, p_ref, c_ref, wt_ref, wb_ref, b_ref,
                wa_ref, wbb_ref, wc_ref, bl1_ref, wl2_ref, bl2_ref, o_ref):
  cnt = jnp.sum(c_ref[...], axis=0)[:, 0:1]
  agg = jnp.sum(p_ref[...], axis=0)
  num = cnt * (jnp.dot(x1_ref[...], wt_ref[...],
                       preferred_element_type=jnp.float32) + b_ref[...])
  num = num + jnp.dot(agg, wb_ref[...], preferred_element_type=jnp.float32)
  x2 = jnp.maximum(num / jnp.maximum(cnt, 1.0), 0.0)
  h = (jnp.dot(x_ref[...], wa_ref[...], preferred_element_type=jnp.float32)
       + jnp.dot(x1_ref[...], wbb_ref[...], preferred_element_type=jnp.float32)
       + jnp.dot(x2, wc_ref[...], preferred_element_type=jnp.float32)
       + bl1_ref[...])
  h = jnp.maximum(h, 0.0)
  logits = jnp.dot(h, wl2_ref[...], preferred_element_type=jnp.float32) + bl2_ref[...]
  o_ref[...] = jax.nn.sigmoid(logits)


def _row_spec(shape):
  return pl.BlockSpec(shape, lambda i: (i, 0))


def _part_spec(shape):
  return pl.BlockSpec(shape, lambda i: (0, i, 0))


def _full_spec(shape):
  return pl.BlockSpec(shape, lambda i: (0,) * len(shape))


def kernel(x, edge_index, W1, b1, W2, b2, Wl1, bl1, Wl2, bl2):
  rows = edge_index[0]
  cols = edge_index[1]
  pad = E_PAD - N_EDGES
  pad_rows = N_NODES + (jnp.arange(pad, dtype=jnp.int32) % (N_PAD - N_NODES))
  pad_cols = jnp.arange(pad, dtype=jnp.int32) % N_NODES
  rows_p = jnp.concatenate([rows, pad_rows])
  cols_p = jnp.concatenate([cols, pad_cols])
  idx = [
      rows_p[:E_SPLIT].reshape(N_TILES, N_CHUNKS0, CHUNK),
      cols_p[:E_SPLIT].reshape(N_TILES, N_CHUNKS0, CHUNK),
  ]
  if N_ACT == 2:
    idx += [
        rows_p[E_SPLIT:].reshape(N_TILES, N_CHUNKS1, CHUNK),
        cols_p[E_SPLIT:].reshape(N_TILES, N_CHUNKS1, CHUNK),
    ]

  sc_pass1 = _make_sc_pass(True)
  sc_pass2 = _make_sc_pass(False)

  p1, c1 = sc_pass1(x, *idx)
  grid = (N_NODES // BLK,)

  x1 = pl.pallas_call(
      _conv_body,
      grid=grid,
      in_specs=[
          _row_spec((BLK, D)),
          _part_spec((N_ACT, BLK, D)),
          _part_spec((N_ACT, BLK, LANES)),
          _full_spec((D, D)),
          _full_spec((D, D)),
          _full_spec((1, D)),
      ],
      out_specs=_row_spec((BLK, D)),
      out_shape=jax.ShapeDtypeStruct((N_NODES, D), jnp.float32),
  )(x, p1, c1, W1[:D], W1[D:], b1.reshape(1, D))

  p2 = sc_pass2(x1, *idx)

  out = pl.pallas_call(
      _final_body,
      grid=grid,
      in_specs=[
          _row_spec((BLK, D)),
          _row_spec((BLK, D)),
          _part_spec((N_ACT, BLK, D)),
          _part_spec((N_ACT, BLK, LANES)),
          _full_spec((D, D)),
          _full_spec((D, D)),
          _full_spec((1, D)),
          _full_spec((D, D)),
          _full_spec((D, D)),
          _full_spec((D, D)),
          _full_spec((1, D)),
          _full_spec((D, D)),
          _full_spec((1, D)),
      ],
      out_specs=_row_spec((BLK, D)),
      out_shape=jax.ShapeDtypeStruct((N_NODES, D), jnp.float32),
  )(x, x1, p2, c1, W2[:D], W2[D:], b2.reshape(1, D),
    Wl1[:D], Wl1[D:2 * D], Wl1[2 * D:], bl1.reshape(1, D),
    Wl2, bl2.reshape(1, D))

  return out


# speedup vs baseline: 3.3708x; 1.4210x over previous
import functools

import jax
import jax.numpy as jnp
from jax import lax
from jax.experimental import pallas as pl
from jax.experimental.pallas import tpu as pltpu
from jax.experimental.pallas import tpu_sc as plsc

N_NODES = 10000
N_EDGES = 320000
D = 128
LANES = 16

N_TILES = 16
CHUNK = 128
GROUP = 8
G0 = 10
G1 = 10
N_CHUNKS0 = G0 * GROUP
N_CHUNKS1 = G1 * GROUP
E_SPLIT = N_TILES * N_CHUNKS0 * CHUNK
E_PAD = N_TILES * (N_CHUNKS0 + N_CHUNKS1) * CHUNK
N_ACT = 1 if G1 == 0 else 2
N_PAD = 10240
ROWS_PER_TILE = N_PAD // N_TILES


def _sc_body(with_cnt, x_hbm, *refs):
  idx_hbm = refs[:2 * N_ACT]
  refs = refs[2 * N_ACT:]
  if with_cnt:
    out_hbm, cnt_hbm, agg_sh, cnt_sh, cbuf, rbuf, gbuf, gbuf2, gsem, obuf = refs
  else:
    out_hbm, agg_sh, cbuf, rbuf, gbuf, gbuf2, gsem = refs

  c = lax.axis_index("c")
  s = lax.axis_index("s")
  slab = s * ROWS_PER_TILE
  gbufs = (gbuf, gbuf2)

  def core_work(core_id, rows_hbm, cols_hbm, n_groups):
    def zrow(r, _):
      for k in range(D // LANES):
        gbuf[r, pl.ds(k * LANES, LANES)] = jnp.zeros((LANES,), jnp.float32)
      return 0
    lax.fori_loop(0, CHUNK, zrow, 0)
    for k in range(ROWS_PER_TILE // CHUNK):
      pltpu.sync_copy(gbuf, agg_sh.at[pl.ds(slab + k * CHUNK, CHUNK)])

    if with_cnt:
      def zrow2(r, _):
        obuf[r] = jnp.zeros((LANES,), jnp.float32)
        return 0
      lax.fori_loop(0, CHUNK, zrow2, 0)
      for k in range(ROWS_PER_TILE // CHUNK):
        pltpu.sync_copy(obuf, cnt_sh.at[pl.ds(slab + k * CHUNK, CHUNK)])
      def orow(r, _):
        obuf[r] = jnp.ones((LANES,), jnp.float32)
        return 0
      lax.fori_loop(0, CHUNK, orow, 0)

    plsc.subcore_barrier()

    def group(g, _):
      pltpu.sync_copy(cols_hbm.at[s, pl.ds(g * GROUP, GROUP)], cbuf)
      pltpu.sync_copy(rows_hbm.at[s, pl.ds(g * GROUP, GROUP)], rbuf)

      h = pltpu.async_copy(x_hbm.at[cbuf.at[0]], gbufs[0], gsem)
      for j in range(GROUP):
        h.wait()
        if j + 1 < GROUP:
          h = pltpu.async_copy(x_hbm.at[cbuf.at[j + 1]], gbufs[(j + 1) % 2],
                               gsem)
        pltpu.sync_copy(gbufs[j % 2], agg_sh.at[rbuf.at[j]], add=True)
        if with_cnt:
          pltpu.sync_copy(obuf, cnt_sh.at[rbuf.at[j]], add=True)
      return 0
    lax.fori_loop(0, n_groups, group, 0)

    plsc.subcore_barrier()
    pltpu.sync_copy(agg_sh.at[pl.ds(slab, ROWS_PER_TILE)],
                    out_hbm.at[core_id, pl.ds(slab, ROWS_PER_TILE)])
    if with_cnt:
      pltpu.sync_copy(cnt_sh.at[pl.ds(slab, ROWS_PER_TILE)],
                      cnt_hbm.at[core_id, pl.ds(slab, ROWS_PER_TILE)])

  @pl.when(c == 0)
  def _():
    core_work(0, idx_hbm[0], idx_hbm[1], G0)

  if N_ACT == 2:
    @pl.when(c == 1)
    def _():
      core_work(1, idx_hbm[2], idx_hbm[3], G1)


def _make_sc_pass(with_cnt):
  out_types = [jax.ShapeDtypeStruct((N_ACT, N_PAD, D), jnp.float32)]
  scratch = [pltpu.VMEM_SHARED((N_PAD, D), jnp.float32)]
  if with_cnt:
    out_types.append(jax.ShapeDtypeStruct((N_ACT, N_PAD, LANES), jnp.float32))
    scratch.append(pltpu.VMEM_SHARED((N_PAD, LANES), jnp.float32))
  scratch += [
      pltpu.VMEM((GROUP, CHUNK), jnp.int32),
      pltpu.VMEM((GROUP, CHUNK), jnp.int32),
      pltpu.VMEM((CHUNK, D), jnp.float32),
      pltpu.VMEM((CHUNK, D), jnp.float32),
      pltpu.SemaphoreType.DMA,
  ]
  if with_cnt:
    scratch.append(pltpu.VMEM((CHUNK, LANES), jnp.float32))
  mesh = plsc.VectorSubcoreMesh(core_axis_name="c", subcore_axis_name="s")
  return pl.kernel(
      functools.partial(_sc_body, with_cnt),
      out_type=tuple(out_types) if with_cnt else out_types[0],
      mesh=mesh,
      scratch_types=scratch,
      compiler_params=pltpu.CompilerParams(use_tc_tiling_on_sc=False),
  )


BLK = 2000


def _conv_body(x_ref, p_ref, c_ref, wt_ref, wb_ref, b_ref, o_ref):
  cnt = jnp.sum(c_ref[...], axis=0)[:, 0:1]
  agg = jnp.sum(p_ref[...], axis=0)
  num = cnt * (jnp.dot(x_ref[...], wt_ref[...],
                       preferred_element_type=jnp.float32) + b_ref[...])
  num = num + jnp.dot(agg, wb_ref[...], preferred_element_type=jnp.float32)
  o_ref[...] = jnp.maximum(num / jnp.maximum(cnt, 1.0), 0.0)


def _final_body(x_ref, x1_ref, p_ref, c_ref, wt_ref, wb_ref, b_ref,
                wa_ref, wbb_ref, wc_ref, bl1_ref, wl2_ref, bl2_ref, o_ref):
  cnt = jnp.sum(c_ref[...], axis=0)[:, 0:1]
  agg = jnp.sum(p_ref[...], axis=0)
  num = cnt * (jnp.dot(x1_ref[...], wt_ref[...],
                       preferred_element_type=jnp.float32) + b_ref[...])
  num = num + jnp.dot(agg, wb_ref[...], preferred_element_type=jnp.float32)
  x2 = jnp.maximum(num / jnp.maximum(cnt, 1.0), 0.0)
  h = (jnp.dot(x_ref[...], wa_ref[...], preferred_element_type=jnp.float32)
       + jnp.dot(x1_ref[...], wbb_ref[...], preferred_element_type=jnp.float32)
       + jnp.dot(x2, wc_ref[...], preferred_element_type=jnp.float32)
       + bl1_ref[...])
  h = jnp.maximum(h, 0.0)
  logits = jnp.dot(h, wl2_ref[...], preferred_element_type=jnp.float32) + bl2_ref[...]
  o_ref[...] = jax.nn.sigmoid(logits)


def _row_spec(shape):
  return pl.BlockSpec(shape, lambda i: (i, 0))


def _part_spec(shape):
  return pl.BlockSpec(shape, lambda i: (0, i, 0))


def _full_spec(shape):
  return pl.BlockSpec(shape, lambda i: (0,) * len(shape))


def kernel(x, edge_index, W1, b1, W2, b2, Wl1, bl1, Wl2, bl2):
  rows = edge_index[0]
  cols = edge_index[1]
  pad = E_PAD - N_EDGES
  pad_rows = N_NODES + (jnp.arange(pad, dtype=jnp.int32) % (N_PAD - N_NODES))
  pad_cols = jnp.arange(pad, dtype=jnp.int32) % N_NODES
  rows_p = jnp.concatenate([rows, pad_rows])
  cols_p = jnp.concatenate([cols, pad_cols])
  idx = [
      rows_p[:E_SPLIT].reshape(N_TILES, N_CHUNKS0, CHUNK),
      cols_p[:E_SPLIT].reshape(N_TILES, N_CHUNKS0, CHUNK),
  ]
  if N_ACT == 2:
    idx += [
        rows_p[E_SPLIT:].reshape(N_TILES, N_CHUNKS1, CHUNK),
        cols_p[E_SPLIT:].reshape(N_TILES, N_CHUNKS1, CHUNK),
    ]

  sc_pass1 = _make_sc_pass(True)
  sc_pass2 = _make_sc_pass(False)

  p1, c1 = sc_pass1(x, *idx)
  grid = (N_NODES // BLK,)

  x1 = pl.pallas_call(
      _conv_body,
      grid=grid,
      in_specs=[
          _row_spec((BLK, D)),
          _part_spec((N_ACT, BLK, D)),
          _part_spec((N_ACT, BLK, LANES)),
          _full_spec((D, D)),
          _full_spec((D, D)),
          _full_spec((1, D)),
      ],
      out_specs=_row_spec((BLK, D)),
      out_shape=jax.ShapeDtypeStruct((N_NODES, D), jnp.float32),
  )(x, p1, c1, W1[:D], W1[D:], b1.reshape(1, D))

  p2 = sc_pass2(x1, *idx)

  out = pl.pallas_call(
      _final_body,
      grid=grid,
      in_specs=[
          _row_spec((BLK, D)),
          _row_spec((BLK, D)),
          _part_spec((N_ACT, BLK, D)),
          _part_spec((N_ACT, BLK, LANES)),
          _full_spec((D, D)),
          _full_spec((D, D)),
          _full_spec((1, D)),
          _full_spec((D, D)),
          _full_spec((D, D)),
          _full_spec((D, D)),
          _full_spec((1, D)),
          _full_spec((D, D)),
          _full_spec((1, D)),
      ],
      out_specs=_row_spec((BLK, D)),
      out_shape=jax.ShapeDtypeStruct((N_NODES, D), jnp.float32),
  )(x, x1, p2, c1, W2[:D], W2[D:], b2.reshape(1, D),
    Wl1[:D], Wl1[D:2 * D], Wl1[2 * D:], bl1.reshape(1, D),
    Wl2, bl2.reshape(1, D))

  return out


# speedup vs baseline: 3.4721x; 1.0300x over previous
import functools

import jax
import jax.numpy as jnp
from jax import lax
from jax.experimental import pallas as pl
from jax.experimental.pallas import tpu as pltpu
from jax.experimental.pallas import tpu_sc as plsc

N_NODES = 10000
N_EDGES = 320000
D = 128
LANES = 16

N_TILES = 16
CHUNK = 128
GROUP = 8
G0 = 10
G1 = 10
N_CHUNKS0 = G0 * GROUP
N_CHUNKS1 = G1 * GROUP
E_SPLIT = N_TILES * N_CHUNKS0 * CHUNK
E_PAD = N_TILES * (N_CHUNKS0 + N_CHUNKS1) * CHUNK
N_ACT = 1 if G1 == 0 else 2
N_PAD = 10240
ROWS_PER_TILE = N_PAD // N_TILES


def _sc_body(with_cnt, x_hbm, *refs):
  idx_hbm = refs[:N_ACT]
  refs = refs[N_ACT:]
  if with_cnt:
    (out_hbm, cnt_hbm, agg_sh, cnt_sh, ibuf, gbuf, gbuf2, gsem, obuf) = refs
  else:
    out_hbm, agg_sh, ibuf, gbuf, gbuf2, gsem = refs

  c = lax.axis_index("c")
  s = lax.axis_index("s")
  slab = s * ROWS_PER_TILE
  gbufs = (gbuf, gbuf2)

  def core_work(core_id, idx_pair_hbm, n_groups):
    def zrow(r, _):
      for k in range(D // LANES):
        gbuf[r, pl.ds(k * LANES, LANES)] = jnp.zeros((LANES,), jnp.float32)
      return 0
    lax.fori_loop(0, CHUNK, zrow, 0)
    for k in range(ROWS_PER_TILE // CHUNK):
      pltpu.sync_copy(gbuf, agg_sh.at[pl.ds(slab + k * CHUNK, CHUNK)])

    if with_cnt:
      def zrow2(r, _):
        obuf[r] = jnp.zeros((LANES,), jnp.float32)
        return 0
      lax.fori_loop(0, CHUNK, zrow2, 0)
      for k in range(ROWS_PER_TILE // CHUNK):
        pltpu.sync_copy(obuf, cnt_sh.at[pl.ds(slab + k * CHUNK, CHUNK)])
      def orow(r, _):
        obuf[r] = jnp.ones((LANES,), jnp.float32)
        return 0
      lax.fori_loop(0, CHUNK, orow, 0)

    plsc.subcore_barrier()

    def group(g, _):
      pltpu.sync_copy(idx_pair_hbm.at[s, g], ibuf)

      h = pltpu.async_copy(x_hbm.at[ibuf.at[GROUP]], gbufs[0], gsem)
      for j in range(GROUP):
        h.wait()
        if j + 1 < GROUP:
          h = pltpu.async_copy(x_hbm.at[ibuf.at[GROUP + j + 1]],
                               gbufs[(j + 1) % 2], gsem)
        pltpu.sync_copy(gbufs[j % 2], agg_sh.at[ibuf.at[j]], add=True)
        if with_cnt:
          pltpu.sync_copy(obuf, cnt_sh.at[ibuf.at[j]], add=True)
      return 0
    lax.fori_loop(0, n_groups, group, 0)

    plsc.subcore_barrier()
    pltpu.sync_copy(agg_sh.at[pl.ds(slab, ROWS_PER_TILE)],
                    out_hbm.at[core_id, pl.ds(slab, ROWS_PER_TILE)])
    if with_cnt:
      pltpu.sync_copy(cnt_sh.at[pl.ds(slab, ROWS_PER_TILE)],
                      cnt_hbm.at[core_id, pl.ds(slab, ROWS_PER_TILE)])

  @pl.when(c == 0)
  def _():
    core_work(0, idx_hbm[0], G0)

  if N_ACT == 2:
    @pl.when(c == 1)
    def _():
      core_work(1, idx_hbm[1], G1)


def _make_sc_pass(with_cnt):
  out_types = [jax.ShapeDtypeStruct((N_ACT, N_PAD, D), jnp.float32)]
  scratch = [pltpu.VMEM_SHARED((N_PAD, D), jnp.float32)]
  if with_cnt:
    out_types.append(jax.ShapeDtypeStruct((N_ACT, N_PAD, LANES), jnp.float32))
    scratch.append(pltpu.VMEM_SHARED((N_PAD, LANES), jnp.float32))
  scratch += [
      pltpu.VMEM((2 * GROUP, CHUNK), jnp.int32),
      pltpu.VMEM((CHUNK, D), jnp.float32),
      pltpu.VMEM((CHUNK, D), jnp.float32),
      pltpu.SemaphoreType.DMA,
  ]
  if with_cnt:
    scratch.append(pltpu.VMEM((CHUNK, LANES), jnp.float32))
  mesh = plsc.VectorSubcoreMesh(core_axis_name="c", subcore_axis_name="s")
  return pl.kernel(
      functools.partial(_sc_body, with_cnt),
      out_type=tuple(out_types) if with_cnt else out_types[0],
      mesh=mesh,
      scratch_types=scratch,
      compiler_params=pltpu.CompilerParams(use_tc_tiling_on_sc=False),
  )


BLK = 2000


def _conv_body(x_ref, p_ref, c_ref, wt_ref, wb_ref, b_ref, o_ref):
  cnt = jnp.sum(c_ref[...], axis=0)[:, 0:1]
  agg = jnp.sum(p_ref[...], axis=0)
  num = cnt * (jnp.dot(x_ref[...], wt_ref[...],
                       preferred_element_type=jnp.float32) + b_ref[...])
  num = num + jnp.dot(agg, wb_ref[...], preferred_element_type=jnp.float32)
  o_ref[...] = jnp.maximum(num / jnp.maximum(cnt, 1.0), 0.0)


def _final_body(x_ref, x1_ref, p_ref, c_ref, wt_ref, wb_ref, b_ref,
                wa_ref, wbb_ref, wc_ref, bl1_ref, wl2_ref, bl2_ref, o_ref):
  cnt = jnp.sum(c_ref[...], axis=0)[:, 0:1]
  agg = jnp.sum(p_ref[...], axis=0)
  num = cnt * (jnp.dot(x1_ref[...], wt_ref[...],
                       preferred_element_type=jnp.float32) + b_ref[...])
  num = num + jnp.dot(agg, wb_ref[...], preferred_element_type=jnp.float32)
  x2 = jnp.maximum(num / jnp.maximum(cnt, 1.0), 0.0)
  h = (jnp.dot(x_ref[...], wa_ref[...], preferred_element_type=jnp.float32)
       + jnp.dot(x1_ref[...], wbb_ref[...], preferred_element_type=jnp.float32)
       + jnp.dot(x2, wc_ref[...], preferred_element_type=jnp.float32)
       + bl1_ref[...])
  h = jnp.maximum(h, 0.0)
  logits = jnp.dot(h, wl2_ref[...], preferred_element_type=jnp.float32) + bl2_ref[...]
  o_ref[...] = jax.nn.sigmoid(logits)


def _row_spec(shape):
  return pl.BlockSpec(shape, lambda i: (i, 0))


def _part_spec(shape):
  return pl.BlockSpec(shape, lambda i: (0, i, 0))


def _full_spec(shape):
  return pl.BlockSpec(shape, lambda i: (0,) * len(shape))


def kernel(x, edge_index, W1, b1, W2, b2, Wl1, bl1, Wl2, bl2):
  rows = edge_index[0]
  cols = edge_index[1]
  pad = E_PAD - N_EDGES
  pad_rows = N_NODES + (jnp.arange(pad, dtype=jnp.int32) % (N_PAD - N_NODES))
  pad_cols = jnp.arange(pad, dtype=jnp.int32) % N_NODES
  rows_p = jnp.concatenate([rows, pad_rows])
  cols_p = jnp.concatenate([cols, pad_cols])
  def pack_idx(r, c_, n_groups):
    r4 = r.reshape(N_TILES, n_groups, GROUP, CHUNK)
    c4 = c_.reshape(N_TILES, n_groups, GROUP, CHUNK)
    return jnp.concatenate([r4, c4], axis=2)

  idx = [pack_idx(rows_p[:E_SPLIT], cols_p[:E_SPLIT], G0)]
  if N_ACT == 2:
    idx.append(pack_idx(rows_p[E_SPLIT:], cols_p[E_SPLIT:], G1))

  sc_pass1 = _make_sc_pass(True)
  sc_pass2 = _make_sc_pass(False)

  p1, c1 = sc_pass1(x, *idx)
  grid = (N_NODES // BLK,)

  x1 = pl.pallas_call(
      _conv_body,
      grid=grid,
      in_specs=[
          _row_spec((BLK, D)),
          _part_spec((N_ACT, BLK, D)),
          _part_spec((N_ACT, BLK, LANES)),
          _full_spec((D, D)),
          _full_spec((D, D)),
          _full_spec((1, D)),
      ],
      out_specs=_row_spec((BLK, D)),
      out_shape=jax.ShapeDtypeStruct((N_NODES, D), jnp.float32),
  )(x, p1, c1, W1[:D], W1[D:], b1.reshape(1, D))

  p2 = sc_pass2(x1, *idx)

  out = pl.pallas_call(
      _final_body,
      grid=grid,
      in_specs=[
          _row_spec((BLK, D)),
          _row_spec((BLK, D)),
          _part_spec((N_ACT, BLK, D)),
          _part_spec((N_ACT, BLK, LANES)),
          _full_spec((D, D)),
          _full_spec((D, D)),
          _full_spec((1, D)),
          _full_spec((D, D)),
          _full_spec((D, D)),
          _full_spec((D, D)),
          _full_spec((1, D)),
          _full_spec((D, D)),
          _full_spec((1, D)),
      ],
      out_specs=_row_spec((BLK, D)),
      out_shape=jax.ShapeDtypeStruct((N_NODES, D), jnp.float32),
  )(x, x1, p2, c1, W2[:D], W2[D:], b2.reshape(1, D),
    Wl1[:D], Wl1[D:2 * D], Wl1[2 * D:], bl1.reshape(1, D),
    Wl2, bl2.reshape(1, D))

  return out


# speedup vs baseline: 3.5851x; 1.0325x over previous
import functools

import jax
import jax.numpy as jnp
from jax import lax
from jax.experimental import pallas as pl
from jax.experimental.pallas import tpu as pltpu
from jax.experimental.pallas import tpu_sc as plsc

N_NODES = 10000
N_EDGES = 320000
D = 128
LANES = 16

N_TILES = 16
CHUNK = 128
GROUP = 16
G0 = 5
G1 = 5
N_CHUNKS0 = G0 * GROUP
N_CHUNKS1 = G1 * GROUP
E_SPLIT = N_TILES * N_CHUNKS0 * CHUNK
E_PAD = N_TILES * (N_CHUNKS0 + N_CHUNKS1) * CHUNK
N_ACT = 1 if G1 == 0 else 2
N_PAD = 10224
ROWS_PER_TILE = N_PAD // N_TILES


def _sc_body(with_cnt, x_hbm, *refs):
  idx_hbm = refs[:N_ACT]
  refs = refs[N_ACT:]
  if with_cnt:
    (out_hbm, cnt_hbm, agg_sh, cnt_sh, ibuf, gbuf, gbuf2, gsem, obuf) = refs
  else:
    out_hbm, agg_sh, ibuf, gbuf, gbuf2, gsem = refs

  c = lax.axis_index("c")
  s = lax.axis_index("s")
  slab = s * ROWS_PER_TILE
  gbufs = (gbuf, gbuf2)

  def core_work(core_id, idx_pair_hbm, n_groups):
    def zrow(r, _):
      for k in range(D // LANES):
        gbuf[r, pl.ds(k * LANES, LANES)] = jnp.zeros((LANES,), jnp.float32)
      return 0
    lax.fori_loop(0, CHUNK, zrow, 0)
    off = 0
    while off < ROWS_PER_TILE:
      n = min(CHUNK, ROWS_PER_TILE - off)
      pltpu.sync_copy(gbuf.at[pl.ds(0, n)],
                      agg_sh.at[pl.ds(slab + off, n)])
      off += n

    if with_cnt:
      def zrow2(r, _):
        obuf[r] = jnp.zeros((LANES,), jnp.float32)
        return 0
      lax.fori_loop(0, CHUNK, zrow2, 0)
      off = 0
      while off < ROWS_PER_TILE:
        n = min(CHUNK, ROWS_PER_TILE - off)
        pltpu.sync_copy(obuf.at[pl.ds(0, n)],
                        cnt_sh.at[pl.ds(slab + off, n)])
        off += n
      def orow(r, _):
        obuf[r] = jnp.ones((LANES,), jnp.float32)
        return 0
      lax.fori_loop(0, CHUNK, orow, 0)

    plsc.subcore_barrier()

    def group(g, _):
      pltpu.sync_copy(idx_pair_hbm.at[s, g], ibuf)

      h = pltpu.async_copy(x_hbm.at[ibuf.at[GROUP]], gbufs[0], gsem)
      for j in range(GROUP):
        h.wait()
        if j + 1 < GROUP:
          h = pltpu.async_copy(x_hbm.at[ibuf.at[GROUP + j + 1]],
                               gbufs[(j + 1) % 2], gsem)
        pltpu.sync_copy(gbufs[j % 2], agg_sh.at[ibuf.at[j]], add=True)
        if with_cnt:
          pltpu.sync_copy(obuf, cnt_sh.at[ibuf.at[j]], add=True)
      return 0
    lax.fori_loop(0, n_groups, group, 0)

    plsc.subcore_barrier()
    pltpu.sync_copy(agg_sh.at[pl.ds(slab, ROWS_PER_TILE)],
                    out_hbm.at[core_id, pl.ds(slab, ROWS_PER_TILE)])
    if with_cnt:
      pltpu.sync_copy(cnt_sh.at[pl.ds(slab, ROWS_PER_TILE)],
                      cnt_hbm.at[core_id, pl.ds(slab, ROWS_PER_TILE)])

  @pl.when(c == 0)
  def _():
    core_work(0, idx_hbm[0], G0)

  if N_ACT == 2:
    @pl.when(c == 1)
    def _():
      core_work(1, idx_hbm[1], G1)


def _make_sc_pass(with_cnt):
  out_types = [jax.ShapeDtypeStruct((N_ACT, N_PAD, D), jnp.float32)]
  scratch = [pltpu.VMEM_SHARED((N_PAD, D), jnp.float32)]
  if with_cnt:
    out_types.append(jax.ShapeDtypeStruct((N_ACT, N_PAD, LANES), jnp.float32))
    scratch.append(pltpu.VMEM_SHARED((N_PAD, LANES), jnp.float32))
  scratch += [
      pltpu.VMEM((2 * GROUP, CHUNK), jnp.int32),
      pltpu.VMEM((CHUNK, D), jnp.float32),
      pltpu.VMEM((CHUNK, D), jnp.float32),
      pltpu.SemaphoreType.DMA,
  ]
  if with_cnt:
    scratch.append(pltpu.VMEM((CHUNK, LANES), jnp.float32))
  mesh = plsc.VectorSubcoreMesh(core_axis_name="c", subcore_axis_name="s")
  return pl.kernel(
      functools.partial(_sc_body, with_cnt),
      out_type=tuple(out_types) if with_cnt else out_types[0],
      mesh=mesh,
      scratch_types=scratch,
      compiler_params=pltpu.CompilerParams(use_tc_tiling_on_sc=False),
  )


BLK = 2000


def _conv_body(x_ref, p_ref, c_ref, wt_ref, wb_ref, b_ref, o_ref):
  cnt = jnp.sum(c_ref[...], axis=0)[:, 0:1]
  agg = jnp.sum(p_ref[...], axis=0)
  num = cnt * (jnp.dot(x_ref[...], wt_ref[...],
                       preferred_element_type=jnp.float32) + b_ref[...])
  num = num + jnp.dot(agg, wb_ref[...], preferred_element_type=jnp.float32)
  o_ref[...] = jnp.maximum(num / jnp.maximum(cnt, 1.0), 0.0)


def _final_body(x_ref, x1_ref, p_ref, c_ref, wt_ref, wb_ref, b_ref,
                wa_ref, wbb_ref, wc_ref, bl1_ref, wl2_ref, bl2_ref, o_ref):
  cnt = jnp.sum(c_ref[...], axis=0)[:, 0:1]
  agg = jnp.sum(p_ref[...], axis=0)
  num = cnt * (jnp.dot(x1_ref[...], wt_ref[...],
                       preferred_element_type=jnp.float32) + b_ref[...])
  num = num + jnp.dot(agg, wb_ref[...], preferred_element_type=jnp.float32)
  x2 = jnp.maximum(num / jnp.maximum(cnt, 1.0), 0.0)
  h = (jnp.dot(x_ref[...], wa_ref[...], preferred_element_type=jnp.float32)
       + jnp.dot(x1_ref[...], wbb_ref[...], preferred_element_type=jnp.float32)
       + jnp.dot(x2, wc_ref[...], preferred_element_type=jnp.float32)
       + bl1_ref[...])
  h = jnp.maximum(h, 0.0)
  logits = jnp.dot(h, wl2_ref[...], preferred_element_type=jnp.float32) + bl2_ref[...]
  o_ref[...] = jax.nn.sigmoid(logits)


def _row_spec(shape):
  return pl.BlockSpec(shape, lambda i: (i, 0))


def _part_spec(shape):
  return pl.BlockSpec(shape, lambda i: (0, i, 0))


def _full_spec(shape):
  return pl.BlockSpec(shape, lambda i: (0,) * len(shape))


def kernel(x, edge_index, W1, b1, W2, b2, Wl1, bl1, Wl2, bl2):
  rows = edge_index[0]
  cols = edge_index[1]
  pad = E_PAD - N_EDGES
  pad_rows = N_NODES + (jnp.arange(pad, dtype=jnp.int32) % (N_PAD - N_NODES))
  pad_cols = jnp.arange(pad, dtype=jnp.int32) % N_NODES
  rows_p = jnp.concatenate([rows, pad_rows])
  cols_p = jnp.concatenate([cols, pad_cols])
  def pack_idx(r, c_, n_groups):
    r4 = r.reshape(N_TILES, n_groups, GROUP, CHUNK)
    c4 = c_.reshape(N_TILES, n_groups, GROUP, CHUNK)
    return jnp.concatenate([r4, c4], axis=2)

  idx = [pack_idx(rows_p[:E_SPLIT], cols_p[:E_SPLIT], G0)]
  if N_ACT == 2:
    idx.append(pack_idx(rows_p[E_SPLIT:], cols_p[E_SPLIT:], G1))

  sc_pass1 = _make_sc_pass(True)
  sc_pass2 = _make_sc_pass(False)

  p1, c1 = sc_pass1(x, *idx)
  grid = (N_NODES // BLK,)

  x1 = pl.pallas_call(
      _conv_body,
      grid=grid,
      in_specs=[
          _row_spec((BLK, D)),
          _part_spec((N_ACT, BLK, D)),
          _part_spec((N_ACT, BLK, LANES)),
          _full_spec((D, D)),
          _full_spec((D, D)),
          _full_spec((1, D)),
      ],
      out_specs=_row_spec((BLK, D)),
      out_shape=jax.ShapeDtypeStruct((N_NODES, D), jnp.float32),
  )(x, p1, c1, W1[:D], W1[D:], b1.reshape(1, D))

  p2 = sc_pass2(x1, *idx)

  out = pl.pallas_call(
      _final_body,
      grid=grid,
      in_specs=[
          _row_spec((BLK, D)),
          _row_spec((BLK, D)),
          _part_spec((N_ACT, BLK, D)),
          _part_spec((N_ACT, BLK, LANES)),
          _full_spec((D, D)),
          _full_spec((D, D)),
          _full_spec((1, D)),
          _full_spec((D, D)),
          _full_spec((D, D)),
          _full_spec((D, D)),
          _full_spec((1, D)),
          _full_spec((D, D)),
          _full_spec((1, D)),
      ],
      out_specs=_row_spec((BLK, D)),
      out_shape=jax.ShapeDtypeStruct((N_NODES, D), jnp.float32),
  )(x, x1, p2, c1, W2[:D], W2[D:], b2.reshape(1, D),
    Wl1[:D], Wl1[D:2 * D], Wl1[2 * D:], bl1.reshape(1, D),
    Wl2, bl2.reshape(1, D))

  return out
